# baseline (speedup 1.0000x reference)
import functools

import jax
import jax.numpy as jnp
from jax import lax
from jax.experimental import pallas as pl
from jax.experimental.pallas import tpu as pltpu
from jax.experimental.pallas import tpu_sc as plsc

NUM_GROUPS = 8
VALID = 2333
PADDED = 2336
HIDDEN = 2048
WORKERS_PER_GROUP = 4
MAIN = VALID // WORKERS_PER_GROUP

R = 16
CHUNK = R * HIDDEN
NBUF = 3
NCH = MAIN // R
TAIL = MAIN - NCH * R


def _unpad_body(inp_hbm, out_hbm, buf0, buf1, buf2, gs0, gs1, gs2,
                ss0, ss1, ss2):
    c = lax.axis_index("c")
    s = lax.axis_index("s")
    wid = s * 2 + c
    g = wid // WORKERS_PER_GROUP
    k = wid % WORKERS_PER_GROUP
    src_base = (g * PADDED + k * MAIN) * HIDDEN
    dst_base = (g * VALID + k * MAIN) * HIDDEN

    bufs = (buf0, buf1, buf2)
    gsem = (gs0, gs1, gs2)
    ssem = (ss0, ss1, ss2)

    def start_gather(i, b):
        return pltpu.async_copy(
            inp_hbm.at[pl.ds(src_base + i * CHUNK, CHUNK)], bufs[b], gsem[b])

    def start_scatter(i, b):
        return pltpu.async_copy(
            bufs[b], out_hbm.at[pl.ds(dst_base + i * CHUNK, CHUNK)], ssem[b])

    hg = [start_gather(b, b) for b in range(NBUF)]
    hs = [None] * NBUF
    for i in range(NCH):
        b = i % NBUF
        hg[b].wait()
        hs[b] = start_scatter(i, b)
        if i + NBUF < NCH:
            hs[b].wait()
            hg[b] = start_gather(i + NBUF, b)
    for i in range(max(0, NCH - NBUF), NCH):
        hs[i % NBUF].wait()

    tsrc = src_base + NCH * CHUNK
    tdst = dst_base + NCH * CHUNK

    @pl.when(k == WORKERS_PER_GROUP - 1)
    def _():
        pltpu.async_copy(
            inp_hbm.at[pl.ds(tsrc, (TAIL + 1) * HIDDEN)],
            buf0.at[pl.ds(0, (TAIL + 1) * HIDDEN)], gs0).wait()
        pltpu.async_copy(
            buf0.at[pl.ds(0, (TAIL + 1) * HIDDEN)],
            out_hbm.at[pl.ds(tdst, (TAIL + 1) * HIDDEN)], ss0).wait()

    @pl.when(k != WORKERS_PER_GROUP - 1)
    def _():
        pltpu.async_copy(
            inp_hbm.at[pl.ds(tsrc, TAIL * HIDDEN)],
            buf0.at[pl.ds(0, TAIL * HIDDEN)], gs0).wait()
        pltpu.async_copy(
            buf0.at[pl.ds(0, TAIL * HIDDEN)],
            out_hbm.at[pl.ds(tdst, TAIL * HIDDEN)], ss0).wait()


_unpad = functools.partial(
    pl.kernel,
    out_type=jax.ShapeDtypeStruct((NUM_GROUPS * VALID * HIDDEN,), jnp.float32),
    mesh=plsc.VectorSubcoreMesh(core_axis_name="c", subcore_axis_name="s"),
    scratch_types=(
        [pltpu.VMEM((CHUNK,), jnp.float32)] * NBUF
        + [pltpu.SemaphoreType.DMA] * (2 * NBUF)
    ),
)(_unpad_body)


@jax.jit
def _run(inp):
    return _unpad(inp.reshape(-1)).reshape(NUM_GROUPS * VALID, HIDDEN)


def kernel(inp, m_splits):
    return _run(inp)
